# baseline (speedup 1.0000x reference)
import functools

import jax
import jax.numpy as jnp
from jax import lax
from jax.experimental import pallas as pl
from jax.experimental.pallas import tpu as pltpu
from jax.experimental.pallas import tpu_sc as plsc

NC = 2
NS = 16
NW = NC * NS
LANES = 16

DW = 8


def _sc_deg_kernel(n_nodes, n_edges, chunk, n_chunks):
    npt = n_nodes // NS

    mesh = plsc.VectorSubcoreMesh(core_axis_name="c", subcore_axis_name="s")

    @functools.partial(
        pl.kernel,
        out_type=jax.ShapeDtypeStruct((NC * n_nodes, DW), jnp.float32),
        mesh=mesh,
        scratch_types=[
            pltpu.VMEM((n_chunks, chunk), jnp.int32),
            pltpu.VMEM((chunk, DW), jnp.float32),
            pltpu.VMEM((npt, DW), jnp.float32),
            pltpu.VMEM_SHARED((n_nodes, DW), jnp.float32),
            pltpu.SemaphoreType.DMA,
        ],
        compiler_params=pltpu.CompilerParams(use_tc_tiling_on_sc=False),
    )
    def deg_kernel(edge_hbm, ones_hbm, zero_hbm, out_hbm,
                   dst_v, ones_v, zbuf, acc_sh, dsem):
        c = lax.axis_index("c")
        s = lax.axis_index("s")

        pltpu.sync_copy(edge_hbm.at[1, s, pl.ds(c * n_chunks, n_chunks)],
                        dst_v)
        pltpu.sync_copy(ones_hbm, ones_v)
        pltpu.sync_copy(zero_hbm.at[pl.ds(s * npt, npt)], zbuf)
        pltpu.sync_copy(zbuf, acc_sh.at[pl.ds(s * npt, npt)])
        plsc.subcore_barrier()

        wave = 8

        def scat(i, _):
            for t in range(wave):
                j = i * wave + t
                pltpu.async_copy(
                    ones_v, acc_sh.at[dst_v.at[j]], dsem, add=True)
            for t in range(wave):
                j = i * wave + t
                pltpu.make_async_copy(
                    ones_v, acc_sh.at[dst_v.at[j]], dsem).wait()
            return 0

        lax.fori_loop(0, n_chunks // wave, scat, 0)
        plsc.subcore_barrier()

        pltpu.sync_copy(acc_sh.at[pl.ds(s * npt, npt)], zbuf)
        pltpu.sync_copy(zbuf, out_hbm.at[pl.ds(c * n_nodes + s * npt, npt)])

    return deg_kernel


def _sc_msg_kernel(n_nodes, dh, chunk, n_chunks):
    npt = n_nodes // NS
    cpb = npt // chunk

    mesh = plsc.VectorSubcoreMesh(core_axis_name="c", subcore_axis_name="s")

    @functools.partial(
        pl.kernel,
        out_type=jax.ShapeDtypeStruct((n_nodes, 2 * dh), jnp.float32),
        mesh=mesh,
        scratch_types=[
            pltpu.VMEM((n_chunks, chunk), jnp.int32),
            pltpu.VMEM((n_chunks, chunk), jnp.int32),
            pltpu.VMEM((chunk, dh), jnp.float32),
            pltpu.VMEM((chunk, dh), jnp.float32),
            pltpu.VMEM((chunk, dh), jnp.float32),
            pltpu.VMEM((chunk, dh), jnp.float32),
            pltpu.VMEM_SHARED((n_nodes, dh), jnp.float32),
            pltpu.SemaphoreType.DMA,
            pltpu.SemaphoreType.DMA,
            pltpu.SemaphoreType.DMA,
            pltpu.SemaphoreType.DMA,
            pltpu.SemaphoreType.DMA,
            pltpu.SemaphoreType.DMA,
            pltpu.SemaphoreType.DMA,
            pltpu.SemaphoreType.DMA,
        ],
        compiler_params=pltpu.CompilerParams(use_tc_tiling_on_sc=False),
    )
    def msg_kernel(edge_hbm, src2_hbm, ylin_hbm, y3_hbm, out_hbm,
                   src_v, dst_v, buf0, buf1, buf2, buf3, acc_sh,
                   gs0, gs1, gs2, gs3, ss0, ss1, ss2, ss3):
        c = lax.axis_index("c")
        s = lax.axis_index("s")

        pltpu.sync_copy(src2_hbm.at[c, s], src_v)
        pltpu.sync_copy(edge_hbm.at[1, s], dst_v)

        for kb in range(cpb):
            base = s * npt + kb * chunk
            pltpu.sync_copy(y3_hbm.at[pl.ds(base, chunk), c], buf0)
            pltpu.sync_copy(buf0, acc_sh.at[pl.ds(base, chunk)])
        plsc.subcore_barrier()

        nbuf = 4
        bufs = (buf0, buf1, buf2, buf3)
        gsems = (gs0, gs1, gs2, gs3)
        ssems = (ss0, ss1, ss2, ss3)

        for b in range(nbuf):
            pltpu.async_copy(ylin_hbm.at[src_v.at[b]], bufs[b], gsems[b])

        def step(i, _):
            for b in range(nbuf):
                j = i * nbuf + b
                pltpu.make_async_copy(
                    ylin_hbm.at[src_v.at[j]], bufs[b], gsems[b]).wait()
                pltpu.async_copy(
                    bufs[b], acc_sh.at[dst_v.at[j]], ssems[b], add=True)
            for b in range(nbuf):
                j = i * nbuf + b
                pltpu.make_async_copy(
                    bufs[b], acc_sh.at[dst_v.at[j]], ssems[b]).wait()
                nj = j + nbuf

                @pl.when(nj < n_chunks)
                def _():
                    pltpu.async_copy(
                        ylin_hbm.at[src_v.at[nj]], bufs[b], gsems[b])
            return 0

        lax.fori_loop(0, n_chunks // nbuf, step, 0)

        plsc.subcore_barrier()

        for kb in range(cpb):
            base = s * npt + kb * chunk
            pltpu.sync_copy(acc_sh.at[pl.ds(base, chunk)], buf0)
            pltpu.sync_copy(
                buf0, out_hbm.at[pl.ds(base, chunk), pl.ds(c * dh, dh)])

    return msg_kernel


def _tc_xw(x, W):
    n, d_in = x.shape
    d_out = W.shape[1]
    blk = 1000
    nblk = n // blk

    def body(x_ref, w_ref, o_ref):
        o_ref[...] = jnp.dot(x_ref[...], w_ref[...],
                             preferred_element_type=jnp.float32)

    return pl.pallas_call(
        body,
        grid=(nblk,),
        in_specs=[
            pl.BlockSpec((blk, d_in), lambda i: (i, 0)),
            pl.BlockSpec((d_in, d_out), lambda i: (0, 0)),
        ],
        out_specs=pl.BlockSpec((blk, d_out), lambda i: (i, 0)),
        out_shape=jax.ShapeDtypeStruct((n, d_out), jnp.float32),
    )(x, W)


def _tc_scale(xw, deg_parts):
    n, d_out = xw.shape
    blk = 1000
    nblk = n // blk

    def body(x_ref, d0_ref, d1_ref, y_ref):
        deg = d0_ref[:, 0:1] + d1_ref[:, 0:1] + 1.0
        y_ref[...] = x_ref[...] * lax.rsqrt(deg)

    return pl.pallas_call(
        body,
        grid=(nblk,),
        in_specs=[
            pl.BlockSpec((blk, d_out), lambda i: (i, 0)),
            pl.BlockSpec((blk, DW), lambda i: (i, 0)),
            pl.BlockSpec((blk, DW), lambda i: (i + nblk, 0)),
        ],
        out_specs=pl.BlockSpec((blk, d_out), lambda i: (i, 0)),
        out_shape=jax.ShapeDtypeStruct((n, d_out), jnp.float32),
    )(xw, deg_parts, deg_parts)


def _tc_final(acc_parts, deg_parts, b, pg_rows, k_keep):
    n, d = acc_parts.shape

    def body(acc_ref, dg_ref, b_ref, pg_ref, out_ref):
        acc = acc_ref[...]
        deg = dg_ref[0:n, 0:1] + dg_ref[n:2 * n, 0:1] + 1.0
        h = jnp.maximum(acc * lax.rsqrt(deg) + b_ref[...], 0.0)

        pg = pg_ref[...]
        p_vec = pg[0:1, :]
        p_norm = jnp.sqrt(jnp.sum(p_vec * p_vec))
        sg = lax.dot_general(pg, h, (((1,), (1,)), ((), ())),
                             preferred_element_type=jnp.float32)
        s_raw = sg[0:1, :]
        g_raw = sg[1:2, :]
        score = jnp.tanh(s_raw / p_norm)
        score = jnp.where(score == 0.0, 0.0, score)

        bits = lax.bitcast_convert_type(score, jnp.uint32)
        ukey = jnp.where(score < 0.0, ~bits, bits | jnp.uint32(0x80000000))

        def tstep(i, t):
            bit = (31 - i).astype(jnp.uint32)
            cand = t | (jnp.uint32(1) << bit)
            cnt = jnp.sum((ukey >= cand).astype(jnp.int32))
            return jnp.where(cnt >= k_keep, cand, t)

        t = lax.fori_loop(0, 32, tstep, jnp.uint32(0))

        mask_gt = ukey > t
        m = k_keep - jnp.sum(mask_gt.astype(jnp.int32))
        tie = ukey == t
        idx = lax.broadcasted_iota(jnp.int32, (1, n), 1)

        def jstep(i, ans):
            cand = ans | (jnp.int32(1) << (13 - i))
            cnt = jnp.sum((tie & (idx <= cand - 1)).astype(jnp.int32))
            return jnp.where(cnt < m, cand, ans)

        jstar = lax.fori_loop(0, 14, jstep, jnp.int32(0))
        sel = mask_gt | (tie & (idx <= jstar))

        g = score * g_raw
        gmax = jnp.max(jnp.where(sel, g, -jnp.inf))
        e = jnp.where(sel, jnp.exp(g - gmax), 0.0)
        zsum = jnp.sum(e)
        w_row = e * score * (1.0 / zsum)
        out_ref[...] = lax.dot_general(
            w_row, h, (((1,), (0,)), ((), ())),
            preferred_element_type=jnp.float32)

    return pl.pallas_call(
        body,
        out_shape=jax.ShapeDtypeStruct((1, d), jnp.float32),
    )(acc_parts, deg_parts, b, pg_rows)


def kernel(x, edge_index, W, b, p, gate_w, gate_b):
    n, d_in = x.shape
    d_out = W.shape[1]
    e = edge_index.shape[1]
    k_keep = -(-4 * n // 5)

    chunk = 125
    nch_deg = e // (NW * chunk)
    nch_msg = e // (NS * chunk)
    edges_msg = edge_index.reshape(2, NS, nch_msg, chunk)

    ones_rows = jnp.ones((chunk, DW), jnp.float32)
    zero_rows = jnp.zeros((n, DW), jnp.float32)

    src2 = (edge_index[0:1] * 2 + jnp.arange(2, dtype=jnp.int32)[:, None]
            ).reshape(2, NS, nch_msg, chunk)

    xw = _tc_xw(x, W)
    deg_parts = _sc_deg_kernel(n, e, chunk, nch_deg)(
        edges_msg, ones_rows, zero_rows)
    y = _tc_scale(xw, deg_parts)
    acc_parts = _sc_msg_kernel(n, d_out // 2, chunk, nch_msg)(
        edges_msg, src2, y.reshape(2 * n, d_out // 2),
        y.reshape(n, 2, d_out // 2))

    pg_rows = jnp.concatenate(
        [p.reshape(1, d_out), gate_w.reshape(1, d_out)], axis=0)
    out = _tc_final(acc_parts, deg_parts,
                    b.reshape(1, d_out), pg_rows, k_keep)
    return out


# speedup vs baseline: 1.2153x; 1.2153x over previous
import functools

import jax
import jax.numpy as jnp
from jax import lax
from jax.experimental import pallas as pl
from jax.experimental.pallas import tpu as pltpu
from jax.experimental.pallas import tpu_sc as plsc

NC = 2
NS = 16
NW = NC * NS
LANES = 16

DW = 8


def _sc_deg_kernel(n_nodes, n_edges, chunk, n_chunks):
    npt = n_nodes // NS

    mesh = plsc.VectorSubcoreMesh(core_axis_name="c", subcore_axis_name="s")

    @functools.partial(
        pl.kernel,
        out_type=jax.ShapeDtypeStruct((NC * n_nodes, DW), jnp.float32),
        mesh=mesh,
        scratch_types=[
            pltpu.VMEM((n_chunks, chunk), jnp.int32),
            pltpu.VMEM((chunk, DW), jnp.float32),
            pltpu.VMEM((npt, DW), jnp.float32),
            pltpu.VMEM_SHARED((n_nodes, DW), jnp.float32),
            pltpu.SemaphoreType.DMA,
        ],
        compiler_params=pltpu.CompilerParams(use_tc_tiling_on_sc=False),
    )
    def deg_kernel(edge_hbm, ones_hbm, zero_hbm, out_hbm,
                   dst_v, ones_v, zbuf, acc_sh, dsem):
        c = lax.axis_index("c")
        s = lax.axis_index("s")

        pltpu.sync_copy(edge_hbm.at[1, s, pl.ds(c * n_chunks, n_chunks)],
                        dst_v)
        pltpu.sync_copy(ones_hbm, ones_v)
        pltpu.sync_copy(zero_hbm.at[pl.ds(s * npt, npt)], zbuf)
        pltpu.sync_copy(zbuf, acc_sh.at[pl.ds(s * npt, npt)])
        plsc.subcore_barrier()

        wave = 8

        def scat(i, _):
            for t in range(wave):
                j = i * wave + t
                pltpu.async_copy(
                    ones_v, acc_sh.at[dst_v.at[j]], dsem, add=True)
            for t in range(wave):
                j = i * wave + t
                pltpu.make_async_copy(
                    ones_v, acc_sh.at[dst_v.at[j]], dsem).wait()
            return 0

        lax.fori_loop(0, n_chunks // wave, scat, 0)
        plsc.subcore_barrier()

        pltpu.sync_copy(acc_sh.at[pl.ds(s * npt, npt)], zbuf)
        pltpu.sync_copy(zbuf, out_hbm.at[pl.ds(c * n_nodes + s * npt, npt)])

    return deg_kernel


def _sc_msg_kernel(n_nodes, dh, chunk, n_chunks):
    npt = n_nodes // NS
    cpb = npt // chunk

    mesh = plsc.VectorSubcoreMesh(core_axis_name="c", subcore_axis_name="s")

    @functools.partial(
        pl.kernel,
        out_type=jax.ShapeDtypeStruct((n_nodes, 2 * dh), jnp.float32),
        mesh=mesh,
        scratch_types=[
            pltpu.VMEM((n_chunks, chunk), jnp.int32),
            pltpu.VMEM((n_chunks, chunk), jnp.int32),
            pltpu.VMEM((chunk, dh), jnp.float32),
            pltpu.VMEM((chunk, dh), jnp.float32),
            pltpu.VMEM((chunk, dh), jnp.float32),
            pltpu.VMEM((chunk, dh), jnp.float32),
            pltpu.VMEM_SHARED((n_nodes, dh), jnp.float32),
            pltpu.SemaphoreType.DMA,
            pltpu.SemaphoreType.DMA,
            pltpu.SemaphoreType.DMA,
            pltpu.SemaphoreType.DMA,
            pltpu.SemaphoreType.DMA,
            pltpu.SemaphoreType.DMA,
            pltpu.SemaphoreType.DMA,
            pltpu.SemaphoreType.DMA,
        ],
        compiler_params=pltpu.CompilerParams(use_tc_tiling_on_sc=False),
    )
    def msg_kernel(edge_hbm, y2_hbm, out_hbm,
                   src_v, dst_v, buf0, buf1, buf2, buf3, acc_sh,
                   gs0, gs1, gs2, gs3, ss0, ss1, ss2, ss3):
        c = lax.axis_index("c")
        s = lax.axis_index("s")

        pltpu.sync_copy(edge_hbm.at[0, s], src_v)
        pltpu.sync_copy(edge_hbm.at[1, s], dst_v)

        for kb in range(cpb):
            base = s * npt + kb * chunk
            pltpu.sync_copy(y2_hbm.at[c, pl.ds(base, chunk)], buf0)
            pltpu.sync_copy(buf0, acc_sh.at[pl.ds(base, chunk)])
        plsc.subcore_barrier()

        nbuf = 4
        bufs = (buf0, buf1, buf2, buf3)
        gsems = (gs0, gs1, gs2, gs3)
        ssems = (ss0, ss1, ss2, ss3)

        def run_pipeline(y_hbm):
            for b in range(nbuf):
                pltpu.async_copy(y_hbm.at[src_v.at[b]], bufs[b], gsems[b])

            def step(i, _):
                for b in range(nbuf):
                    j = i * nbuf + b
                    pltpu.make_async_copy(
                        y_hbm.at[src_v.at[j]], bufs[b], gsems[b]).wait()
                    pltpu.async_copy(
                        bufs[b], acc_sh.at[dst_v.at[j]], ssems[b], add=True)
                for b in range(nbuf):
                    j = i * nbuf + b
                    pltpu.make_async_copy(
                        bufs[b], acc_sh.at[dst_v.at[j]], ssems[b]).wait()
                    nj = j + nbuf

                    @pl.when(nj < n_chunks)
                    def _():
                        pltpu.async_copy(
                            y_hbm.at[src_v.at[nj]], bufs[b], gsems[b])
                return 0

            lax.fori_loop(0, n_chunks // nbuf, step, 0)

        @pl.when(c == 0)
        def _():
            run_pipeline(y2_hbm.at[0])

        @pl.when(c == 1)
        def _():
            run_pipeline(y2_hbm.at[1])

        plsc.subcore_barrier()

        for kb in range(cpb):
            base = s * npt + kb * chunk
            pltpu.sync_copy(acc_sh.at[pl.ds(base, chunk)], buf0)
            pltpu.sync_copy(
                buf0, out_hbm.at[pl.ds(base, chunk), pl.ds(c * dh, dh)])

    return msg_kernel


def _tc_xw(x, W):
    n, d_in = x.shape
    d_out = W.shape[1]
    blk = 1000
    nblk = n // blk

    def body(x_ref, w_ref, o_ref):
        o_ref[...] = jnp.dot(x_ref[...], w_ref[...],
                             preferred_element_type=jnp.float32)

    return pl.pallas_call(
        body,
        grid=(nblk,),
        in_specs=[
            pl.BlockSpec((blk, d_in), lambda i: (i, 0)),
            pl.BlockSpec((d_in, d_out), lambda i: (0, 0)),
        ],
        out_specs=pl.BlockSpec((blk, d_out), lambda i: (i, 0)),
        out_shape=jax.ShapeDtypeStruct((n, d_out), jnp.float32),
    )(x, W)


def _tc_scale(xw, deg_parts):
    n, d_out = xw.shape
    dh = d_out // 2
    blk = 1000
    nblk = n // blk

    def body(x_ref, d0_ref, d1_ref, y_ref):
        deg = d0_ref[:, 0:1] + d1_ref[:, 0:1] + 1.0
        y = x_ref[...] * lax.rsqrt(deg)
        y_ref[0, :, :] = y[:, 0:dh]
        y_ref[1, :, :] = y[:, dh:d_out]

    return pl.pallas_call(
        body,
        grid=(nblk,),
        in_specs=[
            pl.BlockSpec((blk, d_out), lambda i: (i, 0)),
            pl.BlockSpec((blk, DW), lambda i: (i, 0)),
            pl.BlockSpec((blk, DW), lambda i: (i + nblk, 0)),
        ],
        out_specs=pl.BlockSpec((2, blk, dh), lambda i: (0, i, 0)),
        out_shape=jax.ShapeDtypeStruct((2, n, dh), jnp.float32),
    )(xw, deg_parts, deg_parts)


def _tc_final(acc_parts, deg_parts, b, pg_rows, k_keep):
    n, d = acc_parts.shape

    def body(acc_ref, dg_ref, b_ref, pg_ref, out_ref):
        acc = acc_ref[...]
        deg = dg_ref[0:n, 0:1] + dg_ref[n:2 * n, 0:1] + 1.0
        h = jnp.maximum(acc * lax.rsqrt(deg) + b_ref[...], 0.0)

        pg = pg_ref[...]
        p_vec = pg[0:1, :]
        p_norm = jnp.sqrt(jnp.sum(p_vec * p_vec))
        sg = lax.dot_general(pg, h, (((1,), (1,)), ((), ())),
                             preferred_element_type=jnp.float32)
        s_raw = sg[0:1, :]
        g_raw = sg[1:2, :]
        score = jnp.tanh(s_raw / p_norm)
        score = jnp.where(score == 0.0, 0.0, score)

        bits = lax.bitcast_convert_type(score, jnp.uint32)
        ukey = jnp.where(score < 0.0, ~bits, bits | jnp.uint32(0x80000000))

        def tstep(i, t):
            bit = (31 - i).astype(jnp.uint32)
            cand = t | (jnp.uint32(1) << bit)
            cnt = jnp.sum((ukey >= cand).astype(jnp.int32))
            return jnp.where(cnt >= k_keep, cand, t)

        t = lax.fori_loop(0, 32, tstep, jnp.uint32(0))

        mask_gt = ukey > t
        m = k_keep - jnp.sum(mask_gt.astype(jnp.int32))
        tie = ukey == t
        idx = lax.broadcasted_iota(jnp.int32, (1, n), 1)

        def jstep(i, ans):
            cand = ans | (jnp.int32(1) << (13 - i))
            cnt = jnp.sum((tie & (idx <= cand - 1)).astype(jnp.int32))
            return jnp.where(cnt < m, cand, ans)

        jstar = lax.fori_loop(0, 14, jstep, jnp.int32(0))
        sel = mask_gt | (tie & (idx <= jstar))

        g = score * g_raw
        gmax = jnp.max(jnp.where(sel, g, -jnp.inf))
        e = jnp.where(sel, jnp.exp(g - gmax), 0.0)
        zsum = jnp.sum(e)
        w_row = e * score * (1.0 / zsum)
        out_ref[...] = lax.dot_general(
            w_row, h, (((1,), (0,)), ((), ())),
            preferred_element_type=jnp.float32)

    return pl.pallas_call(
        body,
        out_shape=jax.ShapeDtypeStruct((1, d), jnp.float32),
    )(acc_parts, deg_parts, b, pg_rows)


def kernel(x, edge_index, W, b, p, gate_w, gate_b):
    n, d_in = x.shape
    d_out = W.shape[1]
    e = edge_index.shape[1]
    k_keep = -(-4 * n // 5)

    chunk = 125
    nch_deg = e // (NW * chunk)
    nch_msg = e // (NS * chunk)
    edges_msg = edge_index.reshape(2, NS, nch_msg, chunk)

    ones_rows = jnp.ones((chunk, DW), jnp.float32)
    zero_rows = jnp.zeros((n, DW), jnp.float32)

    xw = _tc_xw(x, W)
    deg_parts = _sc_deg_kernel(n, e, chunk, nch_deg)(
        edges_msg, ones_rows, zero_rows)
    y2 = _tc_scale(xw, deg_parts)
    acc_parts = _sc_msg_kernel(n, d_out // 2, chunk, nch_msg)(edges_msg, y2)

    pg_rows = jnp.concatenate(
        [p.reshape(1, d_out), gate_w.reshape(1, d_out)], axis=0)
    out = _tc_final(acc_parts, deg_parts,
                    b.reshape(1, d_out), pg_rows, k_keep)
    return out
